# baseline (speedup 1.0000x reference)
import functools
import jax
import jax.numpy as jnp
from jax import lax
from jax.experimental import pallas as pl
from jax.experimental.pallas import tpu as pltpu
from jax.experimental.pallas import tpu_sc as plsc

VOCAB = 100000
EMBED_DIM = 128
BATCH = 4096
SEQ = 50
TOT = BATCH * SEQ

_NC, _NS = 2, 16
NW = _NC * _NS
PER_W = TOT // NW
NBUF = 3
CHUNK = 320
NSTEP = PER_W // CHUNK


@functools.partial(
    pl.kernel,
    mesh=plsc.VectorSubcoreMesh(core_axis_name="c", subcore_axis_name="s"),
    out_type=jax.ShapeDtypeStruct((TOT, EMBED_DIM), jnp.float32),
    scratch_types=[
        pltpu.VMEM((CHUNK,), jnp.int32),
        pltpu.VMEM((CHUNK,), jnp.int32),
        pltpu.VMEM((CHUNK,), jnp.int32),
        pltpu.VMEM((CHUNK, EMBED_DIM), jnp.float32),
        pltpu.VMEM((CHUNK, EMBED_DIM), jnp.float32),
        pltpu.VMEM((CHUNK, EMBED_DIM), jnp.float32),
        pltpu.SemaphoreType.DMA,
        pltpu.SemaphoreType.DMA,
        pltpu.SemaphoreType.DMA,
        pltpu.SemaphoreType.DMA,
        pltpu.SemaphoreType.DMA,
        pltpu.SemaphoreType.DMA,
        pltpu.SemaphoreType.DMA,
        pltpu.SemaphoreType.DMA,
        pltpu.SemaphoreType.DMA,
    ],
)
def _gather_kernel(idx_hbm, table_hbm, out_hbm,
                   idx0, idx1, idx2, rows0, rows1, rows2,
                   si0, si1, si2, sg0, sg1, sg2, so0, so1, so2):
    wid = lax.axis_index("s") * _NC + lax.axis_index("c")
    base = wid * PER_W
    idxv, rows = [idx0, idx1, idx2], [rows0, rows1, rows2]
    si, sg, so = [si0, si1, si2], [sg0, sg1, sg2], [so0, so1, so2]

    def idx_cp(i):
        b = i % NBUF
        return pltpu.make_async_copy(
            idx_hbm.at[pl.ds(base + i * CHUNK, CHUNK)], idxv[b], si[b])

    def gather_cp(i):
        b = i % NBUF
        return pltpu.make_async_copy(table_hbm.at[idxv[b]], rows[b], sg[b])

    def out_cp(i):
        b = i % NBUF
        return pltpu.make_async_copy(
            rows[b], out_hbm.at[pl.ds(base + i * CHUNK, CHUNK)], so[b])

    for i in range(min(NBUF, NSTEP)):
        idx_cp(i).start()
    idx_cp(0).wait()
    gather_cp(0).start()
    if NSTEP > 1:
        idx_cp(1).wait()
        gather_cp(1).start()
    for i in range(NSTEP):
        gather_cp(i).wait()
        out_cp(i).start()
        if i + NBUF < NSTEP:
            idx_cp(i + NBUF).start()
        if i + 2 < NSTEP:
            if i >= 1:
                out_cp(i - 1).wait()
            idx_cp(i + 2).wait()
            gather_cp(i + 2).start()
    for i in (NSTEP - 3, NSTEP - 2, NSTEP - 1):
        out_cp(i).wait()


def kernel(sentences, table):
    idx = sentences.T.reshape(TOT).astype(jnp.int32)
    out = _gather_kernel(idx, table)
    return out.reshape(SEQ, BATCH, EMBED_DIM).transpose(1, 0, 2)


# speedup vs baseline: 1.0178x; 1.0178x over previous
import functools
import jax
import jax.numpy as jnp
from jax import lax
from jax.experimental import pallas as pl
from jax.experimental.pallas import tpu as pltpu
from jax.experimental.pallas import tpu_sc as plsc

VOCAB = 100000
EMBED_DIM = 128
BATCH = 4096
SEQ = 50
TOT = BATCH * SEQ

_NC, _NS = 2, 16
NW = _NC * _NS
PER_W = TOT // NW
CHUNK = 400
NSTEP = PER_W // CHUNK


@functools.partial(
    pl.kernel,
    mesh=plsc.VectorSubcoreMesh(core_axis_name="c", subcore_axis_name="s"),
    out_type=jax.ShapeDtypeStruct((TOT, EMBED_DIM), jnp.float32),
    scratch_types=[
        pltpu.VMEM((PER_W,), jnp.int32),
        pltpu.VMEM((CHUNK, EMBED_DIM), jnp.float32),
        pltpu.VMEM((CHUNK, EMBED_DIM), jnp.float32),
        pltpu.SemaphoreType.DMA,
        pltpu.SemaphoreType.DMA,
        pltpu.SemaphoreType.DMA,
        pltpu.SemaphoreType.DMA,
    ],
)
def _gather_kernel(idx_hbm, table_hbm, out_hbm,
                   idxv, rows0, rows1, si, sg0, sg1, so0):
    wid = lax.axis_index("s") * _NC + lax.axis_index("c")
    base = wid * PER_W
    rows = [rows0, rows1]
    sg = [sg0, sg1]

    def gather_cp(i):
        b = i % 2
        return pltpu.make_async_copy(
            table_hbm.at[idxv.at[pl.ds(i * CHUNK, CHUNK)]], rows[b], sg[b])

    def out_cp(i):
        b = i % 2
        return pltpu.make_async_copy(
            rows[b], out_hbm.at[pl.ds(base + i * CHUNK, CHUNK)], so0)

    pltpu.make_async_copy(idx_hbm.at[pl.ds(base, PER_W)], idxv, si).start()
    pltpu.make_async_copy(idx_hbm.at[pl.ds(base, PER_W)], idxv, si).wait()
    gather_cp(0).start()
    for i in range(NSTEP):
        gather_cp(i).wait()
        if i + 1 < NSTEP:
            if i >= 1:
                out_cp(i - 1).wait()
            gather_cp(i + 1).start()
        out_cp(i).start()
    out_cp(NSTEP - 2).wait()
    out_cp(NSTEP - 1).wait()


def kernel(sentences, table):
    idx = sentences.T.reshape(TOT).astype(jnp.int32)
    out = _gather_kernel(idx, table)
    return out.reshape(SEQ, BATCH, EMBED_DIM).transpose(1, 0, 2)
